# baseline (speedup 1.0000x reference)
import functools
import numpy as np
import jax
import jax.numpy as jnp
from jax.experimental import pallas as pl
from jax.experimental.pallas import tpu as pltpu
from jax.experimental.pallas import tpu_sc as plsc

_pallas_call = pl.pallas_call

_SUP = 4
_NBR = 20


def _topk_kernel(q_ref, vt_ref, o_ref, *, n_iter, drop_first, v):
    q = q_ref[0]
    vt = vt_ref[0]
    x0 = vt[0:1, :]
    x1 = vt[1:2, :]
    x2 = vt[2:3, :]
    qn = x0 * x0 + x1 * x1 + x2 * x2
    qi = (q[:, 0:1] * q[:, 0:1] + q[:, 1:2] * q[:, 1:2]
          + q[:, 2:3] * q[:, 2:3])
    bf = jnp.bfloat16
    f32 = jnp.float32
    qb = q.astype(bf).astype(f32)
    xb0 = x0.astype(bf).astype(f32)
    xb1 = x1.astype(bf).astype(f32)
    xb2 = x2.astype(bf).astype(f32)
    inner = qb[:, 0:1] * xb0 + qb[:, 1:2] * xb1 + qb[:, 2:3] * xb2
    d = (qn - 2.0 * inner) + qi
    iota = jax.lax.broadcasted_iota(jnp.int32, d.shape, 1)
    big = jnp.float32(jnp.inf)
    cols = []
    for k in range(n_iter):
        mval = jnp.min(d, axis=1, keepdims=True)
        cand = jnp.where(d == mval, iota, v)
        idx = jnp.min(cand, axis=1, keepdims=True)
        if not (drop_first and k == 0):
            cols.append(idx)
        if k < n_iter - 1:
            d = jnp.where(iota == idx, big, d)
    o_ref[0] = jnp.concatenate(cols, axis=1)


def _topk_indices(queries, points, k, drop_first):
    b, m, _ = queries.shape
    v = points.shape[1]
    vt = jnp.pad(jnp.moveaxis(points, 1, 2), ((0, 0), (0, 5), (0, 0)))
    r = min(m, 256)
    kern = functools.partial(
        _topk_kernel, n_iter=k + (1 if drop_first else 0),
        drop_first=drop_first, v=v)
    return _pallas_call(
        kern,
        grid=(b, m // r),
        in_specs=[
            pl.BlockSpec((1, r, 3), lambda bi, i: (bi, i, 0)),
            pl.BlockSpec((1, 8, v), lambda bi, i: (bi, 0, 0)),
        ],
        out_specs=pl.BlockSpec((1, r, k), lambda bi, i: (bi, i, 0)),
        out_shape=jax.ShapeDtypeStruct((b, m, k), jnp.int32),
    )(queries, vt)


def _sc_gather(table, flat_idx):
    n, d = table.shape
    m = flat_idx.shape[0]
    w = 128
    mesh = plsc.VectorSubcoreMesh(core_axis_name="c", subcore_axis_name="s")

    @functools.partial(
        pl.kernel,
        out_type=jax.ShapeDtypeStruct((m, d), table.dtype),
        mesh=mesh)
    def gather_kernel(tab_hbm, i_hbm, o_hbm):
        def body(i_vmem, o_vmem):
            pltpu.sync_copy(tab_hbm.at[i_vmem.at[0]], o_vmem)

        pltpu.emit_pipeline(
            body,
            grid=(m // w,),
            in_specs=[pl.BlockSpec((1, w), lambda i: (0, i))],
            out_specs=[pl.BlockSpec((w, d), lambda i: (i, 0))],
            core_axis_name=("c", "s"),
            dimension_semantics=(pltpu.PARALLEL,),
        )(i_hbm, o_hbm)

    return gather_kernel(table, flat_idx.reshape(1, m))


def _batched_gather(table, idx):
    b, n, d = table.shape
    off = jnp.arange(b, dtype=jnp.int32).reshape((b,) + (1,) * (idx.ndim - 1))
    flat = (idx + off * n).reshape(-1)
    parts = d // 128
    if parts > 1:
        flat = (flat[:, None] * parts
                + jnp.arange(parts, dtype=jnp.int32)[None, :]).reshape(-1)
    out = _sc_gather(table.reshape(b * n * parts, 128), flat)
    return out.reshape(idx.shape + (d,))


def _pack_bf16(x):
    half = x.shape[-1] // 2
    lo = x[..., :half].astype(jnp.bfloat16)
    hi = x[..., half:].astype(jnp.bfloat16)
    pair = jnp.stack([lo, hi], axis=-1)
    return jax.lax.bitcast_convert_type(pair, jnp.float32)


def _unpack_bf16(p):
    u = jax.lax.bitcast_convert_type(p, jnp.uint32)
    lo = jax.lax.bitcast_convert_type(u << 16, jnp.float32)
    hi = jax.lax.bitcast_convert_type(u & jnp.uint32(0xFFFF0000), jnp.float32)
    return jnp.concatenate([lo, hi], axis=-1)


def _combine_kernel(nbr_ref, ctr_ref, dir_ref, *rest, sup, c, relu, surface):
    if surface:
        (o_ref,) = rest
    else:
        sup_ref, cen_ref, o_ref = rest
    nd = nbr_ref[0] - ctr_ref[0][:, None, :]
    norm = jnp.sqrt(jnp.sum(nd * nd, axis=-1, keepdims=True))
    ndn = nd / jnp.maximum(norm, 1e-12)
    dirs = dir_ref[...]
    dn = jnp.sqrt(jnp.sum(dirs * dirs, axis=0, keepdims=True))
    sd = dirs / jnp.maximum(dn, 1e-12)
    theta = (ndn[..., 0:1] * sd[0:1, :][None]
             + ndn[..., 1:2] * sd[1:2, :][None]
             + ndn[..., 2:3] * sd[2:3, :][None])
    theta = jnp.maximum(theta, 0.0)
    act = theta if surface else theta * _unpack_bf16(sup_ref[0])
    msum = jnp.max(act, axis=1)
    out = msum[:, 0:c]
    for s in range(1, sup):
        out = out + msum[:, s * c:(s + 1) * c]
    if not surface:
        out = out + cen_ref[0]
    if relu:
        out = jnp.maximum(out, 0.0)
    o_ref[0] = out


def _conv_combine(nbr_xyz, verts_pad, dirs, sup_g, center, relu, r):
    b, v, nn, _ = nbr_xyz.shape
    sc = dirs.shape[1]
    c = sc // _SUP
    surface = sup_g is None
    kern = functools.partial(
        _combine_kernel, sup=_SUP, c=c, relu=relu, surface=surface)
    in_specs = [
        pl.BlockSpec((1, r, nn, 128), lambda bi, i: (bi, i, 0, 0)),
        pl.BlockSpec((1, r, 128), lambda bi, i: (bi, i, 0)),
        pl.BlockSpec((3, sc), lambda bi, i: (0, 0)),
    ]
    args = [nbr_xyz, verts_pad, dirs]
    if not surface:
        in_specs.append(
            pl.BlockSpec((1, r, nn, sc // 2), lambda bi, i: (bi, i, 0, 0)))
        in_specs.append(pl.BlockSpec((1, r, c), lambda bi, i: (bi, i, 0)))
        args += [sup_g, center]
    return _pallas_call(
        kern,
        grid=(b, v // r),
        in_specs=in_specs,
        out_specs=pl.BlockSpec((1, r, c), lambda bi, i: (bi, i, 0)),
        out_shape=jax.ShapeDtypeStruct((b, v, c), jnp.float32),
    )(*args)


def _linear_kernel(x_ref, w_ref, b_ref, o_ref):
    o_ref[0] = (jnp.dot(x_ref[0], w_ref[...],
                        preferred_element_type=jnp.float32) + b_ref[...])


def _linear(x, w, bias, r=256):
    b, m, k = x.shape
    n = w.shape[1]
    r = min(r, m)
    return _pallas_call(
        _linear_kernel,
        grid=(b, m // r),
        in_specs=[
            pl.BlockSpec((1, r, k), lambda bi, i: (bi, i, 0)),
            pl.BlockSpec((k, n), lambda bi, i: (0, 0)),
            pl.BlockSpec((1, n), lambda bi, i: (0, 0)),
        ],
        out_specs=pl.BlockSpec((1, r, n), lambda bi, i: (bi, i, 0)),
        out_shape=jax.ShapeDtypeStruct((b, m, n), jnp.float32),
    )(x, w, bias.reshape(1, n))


def _pool_max_kernel(g_ref, o_ref):
    o_ref[0] = jnp.max(g_ref[0], axis=1).astype(jnp.float32)


def _pool_max(g, r=128):
    b, p, nn, c = g.shape
    r = min(r, p)
    return _pallas_call(
        _pool_max_kernel,
        grid=(b, p // r),
        in_specs=[pl.BlockSpec((1, r, nn, c), lambda bi, i: (bi, i, 0, 0))],
        out_specs=pl.BlockSpec((1, r, c), lambda bi, i: (bi, i, 0)),
        out_shape=jax.ShapeDtypeStruct((b, p, c), jnp.float32),
    )(g)


def _global_max_kernel(x_ref, o_ref):
    o_ref[0, 0] = jnp.max(x_ref[0], axis=0)


def _global_max(x):
    b, v, c = x.shape
    out = _pallas_call(
        _global_max_kernel,
        grid=(b,),
        in_specs=[pl.BlockSpec((1, v, c), lambda bi: (bi, 0, 0))],
        out_specs=pl.BlockSpec((1, 1, c), lambda bi: (bi, 0, 0)),
        out_shape=jax.ShapeDtypeStruct((b, 1, c), jnp.float32),
    )(x)
    return out[:, 0, :]


def _head_kernel(x_ref, w1_ref, b1_ref, w2_ref, b2_ref, w3_ref, b3_ref, o_ref):
    f32 = jnp.float32
    h = jnp.dot(x_ref[0], w1_ref[...], preferred_element_type=f32) + b1_ref[...]
    h = jnp.maximum(h, 0.0)
    h = jnp.dot(h, w2_ref[...], preferred_element_type=f32) + b2_ref[...]
    h = jnp.maximum(h, 0.0)
    o_ref[0] = jnp.dot(h, w3_ref[...], preferred_element_type=f32) + b3_ref[...]


def _head(x, w1, b1, w2, b2, w3, b3, r=256):
    b, m, k = x.shape
    h1 = w1.shape[1]
    n = w3.shape[1]
    return _pallas_call(
        _head_kernel,
        grid=(b, m // r),
        in_specs=[
            pl.BlockSpec((1, r, k), lambda bi, i: (bi, i, 0)),
            pl.BlockSpec((k, h1), lambda bi, i: (0, 0)),
            pl.BlockSpec((1, h1), lambda bi, i: (0, 0)),
            pl.BlockSpec((h1, h1), lambda bi, i: (0, 0)),
            pl.BlockSpec((1, h1), lambda bi, i: (0, 0)),
            pl.BlockSpec((h1, n), lambda bi, i: (0, 0)),
            pl.BlockSpec((1, n), lambda bi, i: (0, 0)),
        ],
        out_specs=pl.BlockSpec((1, r, n), lambda bi, i: (bi, i, 0)),
        out_shape=jax.ShapeDtypeStruct((b, m, n), jnp.float32),
    )(x, w1, b1.reshape(1, h1), w2, b2.reshape(1, h1), w3, b3.reshape(1, n))


def _pad3(x):
    return jnp.pad(x, ((0, 0), (0, 0), (0, 125)))


def _conv_stage(nbr, vpad, fm_in, w, bias, dirs, nidx, out_c, relu, r):
    fo = _linear(fm_in, w, bias)
    cen = fo[:, :, :out_c]
    sup_g = _batched_gather(_pack_bf16(fo[:, :, out_c:]), nidx)
    return _conv_combine(nbr, vpad, dirs, sup_g, cen, relu, r)


def kernel(vertices, onehot, params):
    preds = [_forward_one(vertices[i:i + 1], onehot[i:i + 1], params)
             for i in range(vertices.shape[0])]
    return jnp.concatenate(preds, axis=0)


def _forward_one(vertices, onehot, params):
    b, v, _ = vertices.shape

    nidx1 = _topk_indices(vertices, vertices, _NBR, True)
    vpad = _pad3(vertices)
    nbr1 = _batched_gather(vpad, nidx1)
    fm0 = _conv_combine(nbr1, vpad, params['d0'], None, None, True, 128)
    fm1 = _conv_stage(nbr1, vpad, fm0, params['w1'], params['b1'],
                      params['d1'], nidx1, 128, True, 128)

    sidx1 = jnp.asarray(np.random.RandomState(1).permutation(v)[:v // 4])
    vq1 = vertices[:, sidx1, :]
    pidx1 = nidx1[:, sidx1, :4]
    fmp1 = _pool_max(_batched_gather(fm1, pidx1))

    v2 = v // 4
    nidx2 = _topk_indices(vq1, vq1, _NBR, True)
    vp1pad = _pad3(vq1)
    nbr2 = _batched_gather(vp1pad, nidx2)
    fm2 = _conv_stage(nbr2, vp1pad, fmp1, params['w2'], params['b2'],
                      params['d2'], nidx2, 256, True, 64)
    fm3 = _conv_stage(nbr2, vp1pad, fm2, params['w3'], params['b3'],
                      params['d3'], nidx2, 256, True, 64)

    sidx2 = jnp.asarray(np.random.RandomState(2).permutation(v2)[:v2 // 4])
    vq2 = vq1[:, sidx2, :]
    pidx2 = nidx2[:, sidx2, :4]
    fmp2 = _pool_max(_batched_gather(fm3, pidx2))

    nidx3 = _topk_indices(vq2, vq2, _NBR, True)
    vp2pad = _pad3(vq2)
    nbr3 = _batched_gather(vp2pad, nidx3)
    fm4 = _conv_stage(nbr3, vp2pad, fmp2, params['w4'], params['b4'],
                      params['d4'], nidx3, 512, False, 32)
    fg = _global_max(fm4)

    near1 = _topk_indices(vertices, vq1, 1, False)
    near2 = _topk_indices(vertices, vq2, 1, False)
    f2u = _batched_gather(fm2, near1)[:, :, 0, :]
    f3u = _batched_gather(fm3, near1)[:, :, 0, :]
    f4u = _batched_gather(fm4, near2)[:, :, 0, :]

    fuse = jnp.concatenate([
        fm0, fm1, f2u, f3u, f4u,
        jnp.broadcast_to(fg[:, None, :], (b, v, fg.shape[-1])),
        jnp.broadcast_to(onehot[:, None, :], (b, v, onehot.shape[-1])),
    ], axis=2)
    k_fuse = fuse.shape[-1]
    k_pad = -k_fuse % 128
    fuse = jnp.pad(fuse, ((0, 0), (0, 0), (0, k_pad)))
    w1t = jnp.pad(params['cw1'].T, ((0, k_pad), (0, 0)))
    return _head(fuse, w1t, params['cb1'], params['cw2'].T, params['cb2'],
                 params['cw3'].T, params['cb3'])


# speedup vs baseline: 1.0709x; 1.0709x over previous
import functools
import numpy as np
import jax
import jax.numpy as jnp
from jax.experimental import pallas as pl
from jax.experimental.pallas import tpu as pltpu
from jax.experimental.pallas import tpu_sc as plsc

_pallas_call = pl.pallas_call

_SUP = 4
_NBR = 20


def _topk_kernel(q_ref, vt_ref, o_ref, *, n_iter, drop_first, v):
    q = q_ref[0]
    vt = vt_ref[0]
    x0 = vt[0:1, :]
    x1 = vt[1:2, :]
    x2 = vt[2:3, :]
    qn = x0 * x0 + x1 * x1 + x2 * x2
    qi = (q[:, 0:1] * q[:, 0:1] + q[:, 1:2] * q[:, 1:2]
          + q[:, 2:3] * q[:, 2:3])
    bf = jnp.bfloat16
    f32 = jnp.float32
    qb = q.astype(bf).astype(f32)
    xb0 = x0.astype(bf).astype(f32)
    xb1 = x1.astype(bf).astype(f32)
    xb2 = x2.astype(bf).astype(f32)
    inner = qb[:, 0:1] * xb0 + qb[:, 1:2] * xb1 + qb[:, 2:3] * xb2
    d = (qn - 2.0 * inner) + qi
    iota = jax.lax.broadcasted_iota(jnp.int32, d.shape, 1)
    big = jnp.float32(jnp.inf)
    cols = []
    for k in range(n_iter):
        mval = jnp.min(d, axis=1, keepdims=True)
        cand = jnp.where(d == mval, iota, v)
        idx = jnp.min(cand, axis=1, keepdims=True)
        if not (drop_first and k == 0):
            cols.append(idx)
        if k < n_iter - 1:
            d = jnp.where(iota == idx, big, d)
    o_ref[0] = jnp.concatenate(cols, axis=1)


def _topk_indices(queries, points, k, drop_first):
    b, m, _ = queries.shape
    v = points.shape[1]
    vt = jnp.pad(jnp.moveaxis(points, 1, 2), ((0, 0), (0, 5), (0, 0)))
    r = min(m, 256)
    kern = functools.partial(
        _topk_kernel, n_iter=k + (1 if drop_first else 0),
        drop_first=drop_first, v=v)
    return _pallas_call(
        kern,
        grid=(b, m // r),
        in_specs=[
            pl.BlockSpec((1, r, 3), lambda bi, i: (bi, i, 0)),
            pl.BlockSpec((1, 8, v), lambda bi, i: (bi, 0, 0)),
        ],
        out_specs=pl.BlockSpec((1, r, k), lambda bi, i: (bi, i, 0)),
        out_shape=jax.ShapeDtypeStruct((b, m, k), jnp.int32),
    )(queries, vt)


def _sc_gather(table, flat_idx):
    n, d = table.shape
    m = flat_idx.shape[0]
    w = 128
    mesh = plsc.VectorSubcoreMesh(core_axis_name="c", subcore_axis_name="s")

    @functools.partial(
        pl.kernel,
        out_type=jax.ShapeDtypeStruct((m, d), table.dtype),
        mesh=mesh)
    def gather_kernel(tab_hbm, i_hbm, o_hbm):
        def body(i_vmem, o_vmem):
            pltpu.sync_copy(tab_hbm.at[i_vmem.at[0]], o_vmem)

        pltpu.emit_pipeline(
            body,
            grid=(m // w,),
            in_specs=[pl.BlockSpec((1, w), lambda i: (0, i))],
            out_specs=[pl.BlockSpec((w, d), lambda i: (i, 0))],
            core_axis_name=("c", "s"),
            dimension_semantics=(pltpu.PARALLEL,),
        )(i_hbm, o_hbm)

    return gather_kernel(table, flat_idx.reshape(1, m))


def _batched_gather(table, idx):
    b, n, d = table.shape
    off = jnp.arange(b, dtype=jnp.int32).reshape((b,) + (1,) * (idx.ndim - 1))
    flat = (idx + off * n).reshape(-1)
    pw = 256 if d % 256 == 0 else 128
    parts = d // pw
    if parts > 1:
        flat = (flat[:, None] * parts
                + jnp.arange(parts, dtype=jnp.int32)[None, :]).reshape(-1)
    out = _sc_gather(table.reshape(b * n * parts, pw), flat)
    return out.reshape(idx.shape + (d,))


def _pack_bf16(x):
    half = x.shape[-1] // 2
    lo = x[..., :half].astype(jnp.bfloat16)
    hi = x[..., half:].astype(jnp.bfloat16)
    pair = jnp.stack([lo, hi], axis=-1)
    return jax.lax.bitcast_convert_type(pair, jnp.float32)


def _unpack_bf16(p):
    u = jax.lax.bitcast_convert_type(p, jnp.uint32)
    lo = jax.lax.bitcast_convert_type(u << 16, jnp.float32)
    hi = jax.lax.bitcast_convert_type(u & jnp.uint32(0xFFFF0000), jnp.float32)
    return jnp.concatenate([lo, hi], axis=-1)


def _combine_kernel(nbr_ref, ctr_ref, dir_ref, *rest, sup, c, relu, surface):
    if surface:
        (o_ref,) = rest
    else:
        sup_ref, cen_ref, o_ref = rest
    nd = nbr_ref[0] - ctr_ref[0][:, None, :]
    norm = jnp.sqrt(jnp.sum(nd * nd, axis=-1, keepdims=True))
    ndn = nd / jnp.maximum(norm, 1e-12)
    dirs = dir_ref[...]
    dn = jnp.sqrt(jnp.sum(dirs * dirs, axis=0, keepdims=True))
    sd = dirs / jnp.maximum(dn, 1e-12)
    theta = (ndn[..., 0:1] * sd[0:1, :][None]
             + ndn[..., 1:2] * sd[1:2, :][None]
             + ndn[..., 2:3] * sd[2:3, :][None])
    theta = jnp.maximum(theta, 0.0)
    act = theta if surface else theta * _unpack_bf16(sup_ref[0])
    msum = jnp.max(act, axis=1)
    out = msum[:, 0:c]
    for s in range(1, sup):
        out = out + msum[:, s * c:(s + 1) * c]
    if not surface:
        out = out + cen_ref[0]
    if relu:
        out = jnp.maximum(out, 0.0)
    o_ref[0] = out


def _conv_combine(nbr_xyz, verts_pad, dirs, sup_g, center, relu, r):
    b, v, nn, _ = nbr_xyz.shape
    sc = dirs.shape[1]
    c = sc // _SUP
    surface = sup_g is None
    kern = functools.partial(
        _combine_kernel, sup=_SUP, c=c, relu=relu, surface=surface)
    in_specs = [
        pl.BlockSpec((1, r, nn, 128), lambda bi, i: (bi, i, 0, 0)),
        pl.BlockSpec((1, r, 128), lambda bi, i: (bi, i, 0)),
        pl.BlockSpec((3, sc), lambda bi, i: (0, 0)),
    ]
    args = [nbr_xyz, verts_pad, dirs]
    if not surface:
        in_specs.append(
            pl.BlockSpec((1, r, nn, sc // 2), lambda bi, i: (bi, i, 0, 0)))
        in_specs.append(pl.BlockSpec((1, r, c), lambda bi, i: (bi, i, 0)))
        args += [sup_g, center]
    return _pallas_call(
        kern,
        grid=(b, v // r),
        in_specs=in_specs,
        out_specs=pl.BlockSpec((1, r, c), lambda bi, i: (bi, i, 0)),
        out_shape=jax.ShapeDtypeStruct((b, v, c), jnp.float32),
    )(*args)


def _linear_kernel(x_ref, w_ref, b_ref, o_ref):
    o_ref[0] = (jnp.dot(x_ref[0], w_ref[...],
                        preferred_element_type=jnp.float32) + b_ref[...])


def _linear(x, w, bias, r=256):
    b, m, k = x.shape
    n = w.shape[1]
    r = min(r, m)
    return _pallas_call(
        _linear_kernel,
        grid=(b, m // r),
        in_specs=[
            pl.BlockSpec((1, r, k), lambda bi, i: (bi, i, 0)),
            pl.BlockSpec((k, n), lambda bi, i: (0, 0)),
            pl.BlockSpec((1, n), lambda bi, i: (0, 0)),
        ],
        out_specs=pl.BlockSpec((1, r, n), lambda bi, i: (bi, i, 0)),
        out_shape=jax.ShapeDtypeStruct((b, m, n), jnp.float32),
    )(x, w, bias.reshape(1, n))


def _pool_max_kernel(g_ref, o_ref):
    o_ref[0] = jnp.max(g_ref[0], axis=1).astype(jnp.float32)


def _pool_max(g, r=128):
    b, p, nn, c = g.shape
    r = min(r, p)
    return _pallas_call(
        _pool_max_kernel,
        grid=(b, p // r),
        in_specs=[pl.BlockSpec((1, r, nn, c), lambda bi, i: (bi, i, 0, 0))],
        out_specs=pl.BlockSpec((1, r, c), lambda bi, i: (bi, i, 0)),
        out_shape=jax.ShapeDtypeStruct((b, p, c), jnp.float32),
    )(g)


def _global_max_kernel(x_ref, o_ref):
    o_ref[0, 0] = jnp.max(x_ref[0], axis=0)


def _global_max(x):
    b, v, c = x.shape
    out = _pallas_call(
        _global_max_kernel,
        grid=(b,),
        in_specs=[pl.BlockSpec((1, v, c), lambda bi: (bi, 0, 0))],
        out_specs=pl.BlockSpec((1, 1, c), lambda bi: (bi, 0, 0)),
        out_shape=jax.ShapeDtypeStruct((b, 1, c), jnp.float32),
    )(x)
    return out[:, 0, :]


def _head_kernel(x_ref, w1_ref, b1_ref, w2_ref, b2_ref, w3_ref, b3_ref, o_ref):
    f32 = jnp.float32
    h = jnp.dot(x_ref[0], w1_ref[...], preferred_element_type=f32) + b1_ref[...]
    h = jnp.maximum(h, 0.0)
    h = jnp.dot(h, w2_ref[...], preferred_element_type=f32) + b2_ref[...]
    h = jnp.maximum(h, 0.0)
    o_ref[0] = jnp.dot(h, w3_ref[...], preferred_element_type=f32) + b3_ref[...]


def _head(x, w1, b1, w2, b2, w3, b3, r=256):
    b, m, k = x.shape
    h1 = w1.shape[1]
    n = w3.shape[1]
    return _pallas_call(
        _head_kernel,
        grid=(b, m // r),
        in_specs=[
            pl.BlockSpec((1, r, k), lambda bi, i: (bi, i, 0)),
            pl.BlockSpec((k, h1), lambda bi, i: (0, 0)),
            pl.BlockSpec((1, h1), lambda bi, i: (0, 0)),
            pl.BlockSpec((h1, h1), lambda bi, i: (0, 0)),
            pl.BlockSpec((1, h1), lambda bi, i: (0, 0)),
            pl.BlockSpec((h1, n), lambda bi, i: (0, 0)),
            pl.BlockSpec((1, n), lambda bi, i: (0, 0)),
        ],
        out_specs=pl.BlockSpec((1, r, n), lambda bi, i: (bi, i, 0)),
        out_shape=jax.ShapeDtypeStruct((b, m, n), jnp.float32),
    )(x, w1, b1.reshape(1, h1), w2, b2.reshape(1, h1), w3, b3.reshape(1, n))


def _pad3(x):
    return jnp.pad(x, ((0, 0), (0, 0), (0, 125)))


def _conv_stage(nbr, vpad, fm_in, w, bias, dirs, nidx, out_c, relu, r):
    fo = _linear(fm_in, w, bias)
    cen = fo[:, :, :out_c]
    sup_g = _batched_gather(_pack_bf16(fo[:, :, out_c:]), nidx)
    return _conv_combine(nbr, vpad, dirs, sup_g, cen, relu, r)


def kernel(vertices, onehot, params):
    preds = [_forward_one(vertices[i:i + 1], onehot[i:i + 1], params)
             for i in range(vertices.shape[0])]
    return jnp.concatenate(preds, axis=0)


def _forward_one(vertices, onehot, params):
    b, v, _ = vertices.shape

    nidx1 = _topk_indices(vertices, vertices, _NBR, True)
    vpad = _pad3(vertices)
    nbr1 = _batched_gather(vpad, nidx1)
    fm0 = _conv_combine(nbr1, vpad, params['d0'], None, None, True, 128)
    fm1 = _conv_stage(nbr1, vpad, fm0, params['w1'], params['b1'],
                      params['d1'], nidx1, 128, True, 128)

    sidx1 = jnp.asarray(np.random.RandomState(1).permutation(v)[:v // 4])
    vq1 = vertices[:, sidx1, :]
    pidx1 = nidx1[:, sidx1, :4]
    fmp1 = _pool_max(_batched_gather(fm1, pidx1))

    v2 = v // 4
    nidx2 = _topk_indices(vq1, vq1, _NBR, True)
    vp1pad = _pad3(vq1)
    nbr2 = _batched_gather(vp1pad, nidx2)
    fm2 = _conv_stage(nbr2, vp1pad, fmp1, params['w2'], params['b2'],
                      params['d2'], nidx2, 256, True, 64)
    fm3 = _conv_stage(nbr2, vp1pad, fm2, params['w3'], params['b3'],
                      params['d3'], nidx2, 256, True, 64)

    sidx2 = jnp.asarray(np.random.RandomState(2).permutation(v2)[:v2 // 4])
    vq2 = vq1[:, sidx2, :]
    pidx2 = nidx2[:, sidx2, :4]
    fmp2 = _pool_max(_batched_gather(fm3, pidx2))

    nidx3 = _topk_indices(vq2, vq2, _NBR, True)
    vp2pad = _pad3(vq2)
    nbr3 = _batched_gather(vp2pad, nidx3)
    fm4 = _conv_stage(nbr3, vp2pad, fmp2, params['w4'], params['b4'],
                      params['d4'], nidx3, 512, False, 32)
    fg = _global_max(fm4)

    near1 = _topk_indices(vertices, vq1, 1, False)
    near2 = _topk_indices(vertices, vq2, 1, False)
    f2u = _batched_gather(fm2, near1)[:, :, 0, :]
    f3u = _batched_gather(fm3, near1)[:, :, 0, :]
    f4u = _batched_gather(fm4, near2)[:, :, 0, :]

    fuse = jnp.concatenate([
        fm0, fm1, f2u, f3u, f4u,
        jnp.broadcast_to(fg[:, None, :], (b, v, fg.shape[-1])),
        jnp.broadcast_to(onehot[:, None, :], (b, v, onehot.shape[-1])),
    ], axis=2)
    k_fuse = fuse.shape[-1]
    k_pad = -k_fuse % 128
    fuse = jnp.pad(fuse, ((0, 0), (0, 0), (0, k_pad)))
    w1t = jnp.pad(params['cw1'].T, ((0, k_pad), (0, 0)))
    return _head(fuse, w1t, params['cb1'], params['cw2'].T, params['cb2'],
                 params['cw3'].T, params['cb3'])


# speedup vs baseline: 1.2146x; 1.1342x over previous
import functools
import numpy as np
import jax
import jax.numpy as jnp
from jax.experimental import pallas as pl
from jax.experimental.pallas import tpu as pltpu
from jax.experimental.pallas import tpu_sc as plsc

_pallas_call = pl.pallas_call

_SUP = 4
_NBR = 20


def _topk_kernel(q_ref, vt_ref, o_ref, *, n_iter, drop_first, v):
    q = q_ref[0]
    vt = vt_ref[0]
    x0 = vt[0:1, :]
    x1 = vt[1:2, :]
    x2 = vt[2:3, :]
    qn = x0 * x0 + x1 * x1 + x2 * x2
    qi = (q[:, 0:1] * q[:, 0:1] + q[:, 1:2] * q[:, 1:2]
          + q[:, 2:3] * q[:, 2:3])
    bf = jnp.bfloat16
    f32 = jnp.float32
    qb = q.astype(bf).astype(f32)
    xb0 = x0.astype(bf).astype(f32)
    xb1 = x1.astype(bf).astype(f32)
    xb2 = x2.astype(bf).astype(f32)
    inner = qb[:, 0:1] * xb0 + qb[:, 1:2] * xb1 + qb[:, 2:3] * xb2
    d = (qn - 2.0 * inner) + qi
    bits = jax.lax.bitcast_convert_type(d, jnp.int32)
    key = jnp.where(d < 0, bits ^ jnp.int32(0x7FFFFFFF), bits)
    iota = jax.lax.broadcasted_iota(jnp.int32, d.shape, 1)
    p = (key & jnp.int32(-2048)) | iota
    cols = []
    for k in range(n_iter):
        m = jnp.min(p, axis=1, keepdims=True)
        if not (drop_first and k == 0):
            cols.append(m & jnp.int32(0x7FF))
        if k < n_iter - 1:
            p = jnp.where(p == m, jnp.int32(0x7FFFFFFF), p)
    o_ref[0] = jnp.concatenate(cols, axis=1)


def _topk_indices(queries, points, k, drop_first):
    b, m, _ = queries.shape
    v = points.shape[1]
    vt = jnp.pad(jnp.moveaxis(points, 1, 2), ((0, 0), (0, 5), (0, 0)))
    r = min(m, 256)
    kern = functools.partial(
        _topk_kernel, n_iter=k + (1 if drop_first else 0),
        drop_first=drop_first, v=v)
    return _pallas_call(
        kern,
        grid=(b, m // r),
        in_specs=[
            pl.BlockSpec((1, r, 3), lambda bi, i: (bi, i, 0)),
            pl.BlockSpec((1, 8, v), lambda bi, i: (bi, 0, 0)),
        ],
        out_specs=pl.BlockSpec((1, r, k), lambda bi, i: (bi, i, 0)),
        out_shape=jax.ShapeDtypeStruct((b, m, k), jnp.int32),
    )(queries, vt)


def _sc_gather(table, flat_idx):
    n, d = table.shape
    m = flat_idx.shape[0]
    w = 128
    mesh = plsc.VectorSubcoreMesh(core_axis_name="c", subcore_axis_name="s")

    @functools.partial(
        pl.kernel,
        out_type=jax.ShapeDtypeStruct((m, d), table.dtype),
        mesh=mesh)
    def gather_kernel(tab_hbm, i_hbm, o_hbm):
        def body(i_vmem, o_vmem):
            pltpu.sync_copy(tab_hbm.at[i_vmem.at[0]], o_vmem)

        pltpu.emit_pipeline(
            body,
            grid=(m // w,),
            in_specs=[pl.BlockSpec((1, w), lambda i: (0, i))],
            out_specs=[pl.BlockSpec((w, d), lambda i: (i, 0))],
            core_axis_name=("c", "s"),
            dimension_semantics=(pltpu.PARALLEL,),
        )(i_hbm, o_hbm)

    return gather_kernel(table, flat_idx.reshape(1, m))


def _batched_gather(table, idx):
    b, n, d = table.shape
    off = jnp.arange(b, dtype=jnp.int32).reshape((b,) + (1,) * (idx.ndim - 1))
    flat = (idx + off * n).reshape(-1)
    pw = 256 if d % 256 == 0 else 128
    parts = d // pw
    if parts > 1:
        flat = (flat[:, None] * parts
                + jnp.arange(parts, dtype=jnp.int32)[None, :]).reshape(-1)
    out = _sc_gather(table.reshape(b * n * parts, pw), flat)
    return out.reshape(idx.shape + (d,))


def _pack_bf16(x):
    half = x.shape[-1] // 2
    lo = x[..., :half].astype(jnp.bfloat16)
    hi = x[..., half:].astype(jnp.bfloat16)
    pair = jnp.stack([lo, hi], axis=-1)
    return jax.lax.bitcast_convert_type(pair, jnp.float32)


def _unpack_bf16(p):
    u = jax.lax.bitcast_convert_type(p, jnp.uint32)
    lo = jax.lax.bitcast_convert_type(u << 16, jnp.float32)
    hi = jax.lax.bitcast_convert_type(u & jnp.uint32(0xFFFF0000), jnp.float32)
    return jnp.concatenate([lo, hi], axis=-1)


def _combine_kernel(nbr_ref, ctr_ref, dir_ref, *rest, sup, c, relu, surface):
    if surface:
        (o_ref,) = rest
    else:
        sup_ref, cen_ref, o_ref = rest
    nd = nbr_ref[0] - ctr_ref[0][:, None, :]
    norm = jnp.sqrt(jnp.sum(nd * nd, axis=-1, keepdims=True))
    ndn = nd / jnp.maximum(norm, 1e-12)
    dirs = dir_ref[...]
    dn = jnp.sqrt(jnp.sum(dirs * dirs, axis=0, keepdims=True))
    sd = dirs / jnp.maximum(dn, 1e-12)
    r, n = nd.shape[0], nd.shape[1]
    scw = sd.shape[1]
    a = ndn[..., :8].reshape(r * n, 8).astype(jnp.bfloat16)
    b = jnp.concatenate([sd, jnp.zeros((5, scw), jnp.float32)],
                        axis=0).astype(jnp.bfloat16)
    theta = jnp.dot(a, b, preferred_element_type=jnp.float32)
    theta = jnp.maximum(theta, 0.0).reshape(r, n, scw)
    act = theta if surface else theta * _unpack_bf16(sup_ref[0])
    msum = jnp.max(act, axis=1)
    out = msum[:, 0:c]
    for s in range(1, sup):
        out = out + msum[:, s * c:(s + 1) * c]
    if not surface:
        out = out + cen_ref[0]
    if relu:
        out = jnp.maximum(out, 0.0)
    o_ref[0] = out


def _conv_combine(nbr_xyz, verts_pad, dirs, sup_g, center, relu, r):
    b, v, nn, _ = nbr_xyz.shape
    sc = dirs.shape[1]
    c = sc // _SUP
    surface = sup_g is None
    kern = functools.partial(
        _combine_kernel, sup=_SUP, c=c, relu=relu, surface=surface)
    in_specs = [
        pl.BlockSpec((1, r, nn, 128), lambda bi, i: (bi, i, 0, 0)),
        pl.BlockSpec((1, r, 128), lambda bi, i: (bi, i, 0)),
        pl.BlockSpec((3, sc), lambda bi, i: (0, 0)),
    ]
    args = [nbr_xyz, verts_pad, dirs]
    if not surface:
        in_specs.append(
            pl.BlockSpec((1, r, nn, sc // 2), lambda bi, i: (bi, i, 0, 0)))
        in_specs.append(pl.BlockSpec((1, r, c), lambda bi, i: (bi, i, 0)))
        args += [sup_g, center]
    return _pallas_call(
        kern,
        grid=(b, v // r),
        in_specs=in_specs,
        out_specs=pl.BlockSpec((1, r, c), lambda bi, i: (bi, i, 0)),
        out_shape=jax.ShapeDtypeStruct((b, v, c), jnp.float32),
    )(*args)


def _linear_kernel(x_ref, w_ref, b_ref, o_ref):
    o_ref[0] = (jnp.dot(x_ref[0], w_ref[...],
                        preferred_element_type=jnp.float32) + b_ref[...])


def _linear(x, w, bias, r=256):
    b, m, k = x.shape
    n = w.shape[1]
    r = min(r, m)
    return _pallas_call(
        _linear_kernel,
        grid=(b, m // r),
        in_specs=[
            pl.BlockSpec((1, r, k), lambda bi, i: (bi, i, 0)),
            pl.BlockSpec((k, n), lambda bi, i: (0, 0)),
            pl.BlockSpec((1, n), lambda bi, i: (0, 0)),
        ],
        out_specs=pl.BlockSpec((1, r, n), lambda bi, i: (bi, i, 0)),
        out_shape=jax.ShapeDtypeStruct((b, m, n), jnp.float32),
    )(x, w, bias.reshape(1, n))


def _pool_max_kernel(g_ref, o_ref):
    o_ref[0] = jnp.max(g_ref[0], axis=1).astype(jnp.float32)


def _pool_max(g, r=128):
    b, p, nn, c = g.shape
    r = min(r, p)
    return _pallas_call(
        _pool_max_kernel,
        grid=(b, p // r),
        in_specs=[pl.BlockSpec((1, r, nn, c), lambda bi, i: (bi, i, 0, 0))],
        out_specs=pl.BlockSpec((1, r, c), lambda bi, i: (bi, i, 0)),
        out_shape=jax.ShapeDtypeStruct((b, p, c), jnp.float32),
    )(g)


def _global_max_kernel(x_ref, o_ref):
    o_ref[0, 0] = jnp.max(x_ref[0], axis=0)


def _global_max(x):
    b, v, c = x.shape
    out = _pallas_call(
        _global_max_kernel,
        grid=(b,),
        in_specs=[pl.BlockSpec((1, v, c), lambda bi: (bi, 0, 0))],
        out_specs=pl.BlockSpec((1, 1, c), lambda bi: (bi, 0, 0)),
        out_shape=jax.ShapeDtypeStruct((b, 1, c), jnp.float32),
    )(x)
    return out[:, 0, :]


def _head_kernel(x_ref, w1_ref, b1_ref, w2_ref, b2_ref, w3_ref, b3_ref, o_ref):
    f32 = jnp.float32
    h = jnp.dot(x_ref[0], w1_ref[...], preferred_element_type=f32) + b1_ref[...]
    h = jnp.maximum(h, 0.0)
    h = jnp.dot(h, w2_ref[...], preferred_element_type=f32) + b2_ref[...]
    h = jnp.maximum(h, 0.0)
    o_ref[0] = jnp.dot(h, w3_ref[...], preferred_element_type=f32) + b3_ref[...]


def _head(x, w1, b1, w2, b2, w3, b3, r=256):
    b, m, k = x.shape
    h1 = w1.shape[1]
    n = w3.shape[1]
    return _pallas_call(
        _head_kernel,
        grid=(b, m // r),
        in_specs=[
            pl.BlockSpec((1, r, k), lambda bi, i: (bi, i, 0)),
            pl.BlockSpec((k, h1), lambda bi, i: (0, 0)),
            pl.BlockSpec((1, h1), lambda bi, i: (0, 0)),
            pl.BlockSpec((h1, h1), lambda bi, i: (0, 0)),
            pl.BlockSpec((1, h1), lambda bi, i: (0, 0)),
            pl.BlockSpec((h1, n), lambda bi, i: (0, 0)),
            pl.BlockSpec((1, n), lambda bi, i: (0, 0)),
        ],
        out_specs=pl.BlockSpec((1, r, n), lambda bi, i: (bi, i, 0)),
        out_shape=jax.ShapeDtypeStruct((b, m, n), jnp.float32),
    )(x, w1, b1.reshape(1, h1), w2, b2.reshape(1, h1), w3, b3.reshape(1, n))


def _pad3(x):
    return jnp.pad(x, ((0, 0), (0, 0), (0, 125)))


def _conv_stage(nbr, vpad, fm_in, w, bias, dirs, nidx, out_c, relu, r):
    fo = _linear(fm_in, w, bias)
    cen = fo[:, :, :out_c]
    sup_g = _batched_gather(_pack_bf16(fo[:, :, out_c:]), nidx)
    return _conv_combine(nbr, vpad, dirs, sup_g, cen, relu, r)


def kernel(vertices, onehot, params):
    preds = [_forward_one(vertices[i:i + 1], onehot[i:i + 1], params)
             for i in range(vertices.shape[0])]
    return jnp.concatenate(preds, axis=0)


def _forward_one(vertices, onehot, params):
    b, v, _ = vertices.shape

    nidx1 = _topk_indices(vertices, vertices, _NBR, True)
    vpad = _pad3(vertices)
    nbr1 = _batched_gather(vpad, nidx1)
    fm0 = _conv_combine(nbr1, vpad, params['d0'], None, None, True, 128)
    fm1 = _conv_stage(nbr1, vpad, fm0, params['w1'], params['b1'],
                      params['d1'], nidx1, 128, True, 128)

    sidx1 = jnp.asarray(np.random.RandomState(1).permutation(v)[:v // 4])
    vq1 = vertices[:, sidx1, :]
    pidx1 = nidx1[:, sidx1, :4]
    fmp1 = _pool_max(_batched_gather(fm1, pidx1))

    v2 = v // 4
    nidx2 = _topk_indices(vq1, vq1, _NBR, True)
    vp1pad = _pad3(vq1)
    nbr2 = _batched_gather(vp1pad, nidx2)
    fm2 = _conv_stage(nbr2, vp1pad, fmp1, params['w2'], params['b2'],
                      params['d2'], nidx2, 256, True, 64)
    fm3 = _conv_stage(nbr2, vp1pad, fm2, params['w3'], params['b3'],
                      params['d3'], nidx2, 256, True, 64)

    sidx2 = jnp.asarray(np.random.RandomState(2).permutation(v2)[:v2 // 4])
    vq2 = vq1[:, sidx2, :]
    pidx2 = nidx2[:, sidx2, :4]
    fmp2 = _pool_max(_batched_gather(fm3, pidx2))

    nidx3 = _topk_indices(vq2, vq2, _NBR, True)
    vp2pad = _pad3(vq2)
    nbr3 = _batched_gather(vp2pad, nidx3)
    fm4 = _conv_stage(nbr3, vp2pad, fmp2, params['w4'], params['b4'],
                      params['d4'], nidx3, 512, False, 32)
    fg = _global_max(fm4)

    near1 = _topk_indices(vertices, vq1, 1, False)
    near2 = _topk_indices(vertices, vq2, 1, False)
    f2u = _batched_gather(fm2, near1)[:, :, 0, :]
    f3u = _batched_gather(fm3, near1)[:, :, 0, :]
    f4u = _batched_gather(fm4, near2)[:, :, 0, :]

    fuse = jnp.concatenate([
        fm0, fm1, f2u, f3u, f4u,
        jnp.broadcast_to(fg[:, None, :], (b, v, fg.shape[-1])),
        jnp.broadcast_to(onehot[:, None, :], (b, v, onehot.shape[-1])),
    ], axis=2)
    k_fuse = fuse.shape[-1]
    k_pad = -k_fuse % 128
    fuse = jnp.pad(fuse, ((0, 0), (0, 0), (0, k_pad)))
    w1t = jnp.pad(params['cw1'].T, ((0, k_pad), (0, 0)))
    return _head(fuse, w1t, params['cb1'], params['cw2'].T, params['cb2'],
                 params['cw3'].T, params['cb3'])


# speedup vs baseline: 1.2280x; 1.0110x over previous
import functools
import numpy as np
import jax
import jax.numpy as jnp
from jax.experimental import pallas as pl
from jax.experimental.pallas import tpu as pltpu
from jax.experimental.pallas import tpu_sc as plsc

_pallas_call = pl.pallas_call

_SUP = 4
_NBR = 20


def _topk_kernel(q_ref, vt_ref, o_ref, *, n_iter, drop_first, v):
    q = q_ref[0]
    vt = vt_ref[0]
    x0 = vt[0:1, :]
    x1 = vt[1:2, :]
    x2 = vt[2:3, :]
    qn = x0 * x0 + x1 * x1 + x2 * x2
    qi = (q[:, 0:1] * q[:, 0:1] + q[:, 1:2] * q[:, 1:2]
          + q[:, 2:3] * q[:, 2:3])
    bf = jnp.bfloat16
    f32 = jnp.float32
    qb = q.astype(bf).astype(f32)
    xb0 = x0.astype(bf).astype(f32)
    xb1 = x1.astype(bf).astype(f32)
    xb2 = x2.astype(bf).astype(f32)
    inner = qb[:, 0:1] * xb0 + qb[:, 1:2] * xb1 + qb[:, 2:3] * xb2
    d = (qn - 2.0 * inner) + qi
    bits = jax.lax.bitcast_convert_type(d, jnp.int32)
    key = jnp.where(d < 0, bits ^ jnp.int32(0x7FFFFFFF), bits)
    iota = jax.lax.broadcasted_iota(jnp.int32, d.shape, 1)
    p = (key & jnp.int32(-2048)) | iota
    cols = []
    for k in range(n_iter):
        m = jnp.min(p, axis=1, keepdims=True)
        if not (drop_first and k == 0):
            cols.append(m & jnp.int32(0x7FF))
        if k < n_iter - 1:
            p = jnp.where(p == m, jnp.int32(0x7FFFFFFF), p)
    o_ref[0] = jnp.concatenate(cols, axis=1)


def _topk_indices(queries, points, k, drop_first):
    b, m, _ = queries.shape
    v = points.shape[1]
    vt = jnp.pad(jnp.moveaxis(points, 1, 2), ((0, 0), (0, 5), (0, 0)))
    r = min(m, 256)
    kern = functools.partial(
        _topk_kernel, n_iter=k + (1 if drop_first else 0),
        drop_first=drop_first, v=v)
    return _pallas_call(
        kern,
        grid=(b, m // r),
        in_specs=[
            pl.BlockSpec((1, r, 3), lambda bi, i: (bi, i, 0)),
            pl.BlockSpec((1, 8, v), lambda bi, i: (bi, 0, 0)),
        ],
        out_specs=pl.BlockSpec((1, r, k), lambda bi, i: (bi, i, 0)),
        out_shape=jax.ShapeDtypeStruct((b, m, k), jnp.int32),
    )(queries, vt)


def _sc_gather(table, flat_idx):
    n, d = table.shape
    m = flat_idx.shape[0]
    w = 128
    mesh = plsc.VectorSubcoreMesh(core_axis_name="c", subcore_axis_name="s")

    @functools.partial(
        pl.kernel,
        out_type=jax.ShapeDtypeStruct((m, d), table.dtype),
        mesh=mesh)
    def gather_kernel(tab_hbm, i_hbm, o_hbm):
        def body(i_vmem, o_vmem):
            pltpu.sync_copy(tab_hbm.at[i_vmem.at[0]], o_vmem)

        pltpu.emit_pipeline(
            body,
            grid=(m // w,),
            in_specs=[pl.BlockSpec((1, w), lambda i: (0, i))],
            out_specs=[pl.BlockSpec((w, d), lambda i: (i, 0))],
            core_axis_name=("c", "s"),
            dimension_semantics=(pltpu.PARALLEL,),
        )(i_hbm, o_hbm)

    return gather_kernel(table, flat_idx.reshape(1, m))


def _batched_gather(table, idx):
    b, n, d = table.shape
    off = jnp.arange(b, dtype=jnp.int32).reshape((b,) + (1,) * (idx.ndim - 1))
    flat = (idx + off * n).reshape(-1)
    pw = 256 if d % 256 == 0 else 128
    parts = d // pw
    if parts > 1:
        flat = (flat[:, None] * parts
                + jnp.arange(parts, dtype=jnp.int32)[None, :]).reshape(-1)
    out = _sc_gather(table.reshape(b * n * parts, pw), flat)
    return out.reshape(idx.shape + (d,))


def _pack_bf16(x):
    half = x.shape[-1] // 2
    lo = x[..., :half].astype(jnp.bfloat16)
    hi = x[..., half:].astype(jnp.bfloat16)
    pair = jnp.stack([lo, hi], axis=-1)
    return jax.lax.bitcast_convert_type(pair, jnp.float32)


def _unpack_bf16(p):
    u = jax.lax.bitcast_convert_type(p, jnp.uint32)
    lo = jax.lax.bitcast_convert_type(u << 16, jnp.float32)
    hi = jax.lax.bitcast_convert_type(u & jnp.uint32(0xFFFF0000), jnp.float32)
    return jnp.concatenate([lo, hi], axis=-1)


def _combine_kernel(nbr_ref, ctr_ref, dir_ref, *rest, sup, c, relu, surface):
    if surface:
        (o_ref,) = rest
    else:
        sup_ref, cen_ref, o_ref = rest
    nd = nbr_ref[0] - ctr_ref[0][:, None, :]
    norm = jnp.sqrt(jnp.sum(nd * nd, axis=-1, keepdims=True))
    ndn = nd / jnp.maximum(norm, 1e-12)
    dirs = dir_ref[...]
    dn = jnp.sqrt(jnp.sum(dirs * dirs, axis=0, keepdims=True))
    sd = dirs / jnp.maximum(dn, 1e-12)
    r, n = nd.shape[0], nd.shape[1]
    scw = sd.shape[1]
    a = ndn[..., :8].reshape(r * n, 8).astype(jnp.bfloat16)
    b = jnp.concatenate([sd, jnp.zeros((5, scw), jnp.float32)],
                        axis=0).astype(jnp.bfloat16)
    theta = jnp.dot(a, b, preferred_element_type=jnp.float32)
    theta = jnp.maximum(theta, 0.0).reshape(r, n, scw)
    act = theta if surface else theta * _unpack_bf16(sup_ref[0])
    msum = jnp.max(act, axis=1)
    out = msum[:, 0:c]
    for s in range(1, sup):
        out = out + msum[:, s * c:(s + 1) * c]
    if not surface:
        out = out + cen_ref[0]
    if relu:
        out = jnp.maximum(out, 0.0)
    o_ref[0] = out


def _conv_combine(nbr_xyz, verts_pad, dirs, sup_g, center, relu, r):
    b, v, nn, _ = nbr_xyz.shape
    sc = dirs.shape[1]
    c = sc // _SUP
    surface = sup_g is None
    kern = functools.partial(
        _combine_kernel, sup=_SUP, c=c, relu=relu, surface=surface)
    in_specs = [
        pl.BlockSpec((1, r, nn, 128), lambda bi, i: (bi, i, 0, 0)),
        pl.BlockSpec((1, r, 128), lambda bi, i: (bi, i, 0)),
        pl.BlockSpec((3, sc), lambda bi, i: (0, 0)),
    ]
    args = [nbr_xyz, verts_pad, dirs]
    if not surface:
        in_specs.append(
            pl.BlockSpec((1, r, nn, sc // 2), lambda bi, i: (bi, i, 0, 0)))
        in_specs.append(pl.BlockSpec((1, r, c), lambda bi, i: (bi, i, 0)))
        args += [sup_g, center]
    return _pallas_call(
        kern,
        grid=(b, v // r),
        in_specs=in_specs,
        out_specs=pl.BlockSpec((1, r, c), lambda bi, i: (bi, i, 0)),
        out_shape=jax.ShapeDtypeStruct((b, v, c), jnp.float32),
    )(*args)


def _linear_kernel(x_ref, w_ref, b_ref, o_ref):
    o_ref[0] = (jnp.dot(x_ref[0], w_ref[...],
                        preferred_element_type=jnp.float32) + b_ref[...])


def _linear(x, w, bias, r=256):
    b, m, k = x.shape
    n = w.shape[1]
    r = min(r, m)
    return _pallas_call(
        _linear_kernel,
        grid=(b, m // r),
        in_specs=[
            pl.BlockSpec((1, r, k), lambda bi, i: (bi, i, 0)),
            pl.BlockSpec((k, n), lambda bi, i: (0, 0)),
            pl.BlockSpec((1, n), lambda bi, i: (0, 0)),
        ],
        out_specs=pl.BlockSpec((1, r, n), lambda bi, i: (bi, i, 0)),
        out_shape=jax.ShapeDtypeStruct((b, m, n), jnp.float32),
    )(x, w, bias.reshape(1, n))


def _pool_max_kernel(g_ref, o_ref):
    o_ref[0] = jnp.max(g_ref[0], axis=1).astype(jnp.float32)


def _pool_max(g, r=128):
    b, p, nn, c = g.shape
    r = min(r, p)
    return _pallas_call(
        _pool_max_kernel,
        grid=(b, p // r),
        in_specs=[pl.BlockSpec((1, r, nn, c), lambda bi, i: (bi, i, 0, 0))],
        out_specs=pl.BlockSpec((1, r, c), lambda bi, i: (bi, i, 0)),
        out_shape=jax.ShapeDtypeStruct((b, p, c), jnp.float32),
    )(g)


def _global_max_kernel(x_ref, o_ref):
    o_ref[0, 0] = jnp.max(x_ref[0], axis=0)


def _global_max(x):
    b, v, c = x.shape
    out = _pallas_call(
        _global_max_kernel,
        grid=(b,),
        in_specs=[pl.BlockSpec((1, v, c), lambda bi: (bi, 0, 0))],
        out_specs=pl.BlockSpec((1, 1, c), lambda bi: (bi, 0, 0)),
        out_shape=jax.ShapeDtypeStruct((b, 1, c), jnp.float32),
    )(x)
    return out[:, 0, :]


def _head_kernel(x_ref, w1_ref, b1_ref, w2_ref, b2_ref, w3_ref, b3_ref, o_ref):
    f32 = jnp.float32
    h = jnp.dot(x_ref[0], w1_ref[...], preferred_element_type=f32) + b1_ref[...]
    h = jnp.maximum(h, 0.0)
    h = jnp.dot(h, w2_ref[...], preferred_element_type=f32) + b2_ref[...]
    h = jnp.maximum(h, 0.0)
    o_ref[0] = jnp.dot(h, w3_ref[...], preferred_element_type=f32) + b3_ref[...]


def _head(x, w1, b1, w2, b2, w3, b3, r=256):
    b, m, k = x.shape
    h1 = w1.shape[1]
    n = w3.shape[1]
    return _pallas_call(
        _head_kernel,
        grid=(b, m // r),
        in_specs=[
            pl.BlockSpec((1, r, k), lambda bi, i: (bi, i, 0)),
            pl.BlockSpec((k, h1), lambda bi, i: (0, 0)),
            pl.BlockSpec((1, h1), lambda bi, i: (0, 0)),
            pl.BlockSpec((h1, h1), lambda bi, i: (0, 0)),
            pl.BlockSpec((1, h1), lambda bi, i: (0, 0)),
            pl.BlockSpec((h1, n), lambda bi, i: (0, 0)),
            pl.BlockSpec((1, n), lambda bi, i: (0, 0)),
        ],
        out_specs=pl.BlockSpec((1, r, n), lambda bi, i: (bi, i, 0)),
        out_shape=jax.ShapeDtypeStruct((b, m, n), jnp.float32),
    )(x, w1, b1.reshape(1, h1), w2, b2.reshape(1, h1), w3, b3.reshape(1, n))


def _pad3(x):
    return jnp.pad(x, ((0, 0), (0, 0), (0, 125)))


def _conv_stage(nbr, vpad, fm_in, w, bias, dirs, nidx, out_c, relu, r):
    fo = _linear(fm_in, w, bias)
    cen = fo[:, :, :out_c]
    sup_g = _batched_gather(_pack_bf16(fo[:, :, out_c:]), nidx)
    return _conv_combine(nbr, vpad, dirs, sup_g, cen, relu, r)


def kernel(vertices, onehot, params):
    return _forward_one(vertices, onehot, params)


def _forward_one(vertices, onehot, params):
    b, v, _ = vertices.shape

    nidx1 = _topk_indices(vertices, vertices, _NBR, True)
    vpad = _pad3(vertices)
    nbr1 = _batched_gather(vpad, nidx1)
    fm0 = _conv_combine(nbr1, vpad, params['d0'], None, None, True, 128)
    fm1 = _conv_stage(nbr1, vpad, fm0, params['w1'], params['b1'],
                      params['d1'], nidx1, 128, True, 128)

    sidx1 = jnp.asarray(np.random.RandomState(1).permutation(v)[:v // 4])
    vq1 = vertices[:, sidx1, :]
    pidx1 = nidx1[:, sidx1, :4]
    fmp1 = _pool_max(_batched_gather(fm1, pidx1))

    v2 = v // 4
    nidx2 = _topk_indices(vq1, vq1, _NBR, True)
    vp1pad = _pad3(vq1)
    nbr2 = _batched_gather(vp1pad, nidx2)
    fm2 = _conv_stage(nbr2, vp1pad, fmp1, params['w2'], params['b2'],
                      params['d2'], nidx2, 256, True, 64)
    fm3 = _conv_stage(nbr2, vp1pad, fm2, params['w3'], params['b3'],
                      params['d3'], nidx2, 256, True, 64)

    sidx2 = jnp.asarray(np.random.RandomState(2).permutation(v2)[:v2 // 4])
    vq2 = vq1[:, sidx2, :]
    pidx2 = nidx2[:, sidx2, :4]
    fmp2 = _pool_max(_batched_gather(fm3, pidx2))

    nidx3 = _topk_indices(vq2, vq2, _NBR, True)
    vp2pad = _pad3(vq2)
    nbr3 = _batched_gather(vp2pad, nidx3)
    fm4 = _conv_stage(nbr3, vp2pad, fmp2, params['w4'], params['b4'],
                      params['d4'], nidx3, 512, False, 32)
    fg = _global_max(fm4)

    near1 = _topk_indices(vertices, vq1, 1, False)
    near2 = _topk_indices(vertices, vq2, 1, False)
    f23u = _batched_gather(jnp.concatenate([fm2, fm3], axis=2),
                           near1)[:, :, 0, :]
    f2u = f23u[:, :, :fm2.shape[2]]
    f3u = f23u[:, :, fm2.shape[2]:]
    f4u = _batched_gather(fm4, near2)[:, :, 0, :]

    fuse = jnp.concatenate([
        fm0, fm1, f2u, f3u, f4u,
        jnp.broadcast_to(fg[:, None, :], (b, v, fg.shape[-1])),
        jnp.broadcast_to(onehot[:, None, :], (b, v, onehot.shape[-1])),
    ], axis=2)
    k_fuse = fuse.shape[-1]
    k_pad = -k_fuse % 128
    fuse = jnp.pad(fuse, ((0, 0), (0, 0), (0, k_pad)))
    w1t = jnp.pad(params['cw1'].T, ((0, k_pad), (0, 0)))
    return _head(fuse, w1t, params['cb1'], params['cw2'].T, params['cb2'],
                 params['cw3'].T, params['cb3'])


# speedup vs baseline: 1.2350x; 1.0057x over previous
import functools
import numpy as np
import jax
import jax.numpy as jnp
from jax.experimental import pallas as pl
from jax.experimental.pallas import tpu as pltpu
from jax.experimental.pallas import tpu_sc as plsc

_pallas_call = pl.pallas_call

_SUP = 4
_NBR = 20


def _topk_kernel(q_ref, vt_ref, o_ref, *, n_iter, drop_first, v):
    q = q_ref[0]
    vt = vt_ref[0]
    x0 = vt[0:1, :]
    x1 = vt[1:2, :]
    x2 = vt[2:3, :]
    qn = x0 * x0 + x1 * x1 + x2 * x2
    qi = (q[:, 0:1] * q[:, 0:1] + q[:, 1:2] * q[:, 1:2]
          + q[:, 2:3] * q[:, 2:3])
    bf = jnp.bfloat16
    f32 = jnp.float32
    qb = q.astype(bf).astype(f32)
    xb0 = x0.astype(bf).astype(f32)
    xb1 = x1.astype(bf).astype(f32)
    xb2 = x2.astype(bf).astype(f32)
    inner = qb[:, 0:1] * xb0 + qb[:, 1:2] * xb1 + qb[:, 2:3] * xb2
    d = (qn - 2.0 * inner) + qi
    bits = jax.lax.bitcast_convert_type(d, jnp.int32)
    key = jnp.where(d < 0, bits ^ jnp.int32(0x7FFFFFFF), bits)
    iota = jax.lax.broadcasted_iota(jnp.int32, d.shape, 1)
    p = (key & jnp.int32(-2048)) | iota
    cols = []
    for k in range(n_iter):
        m = jnp.min(p, axis=1, keepdims=True)
        if not (drop_first and k == 0):
            cols.append(m & jnp.int32(0x7FF))
        if k < n_iter - 1:
            p = jnp.where(p == m, jnp.int32(0x7FFFFFFF), p)
    o_ref[0] = jnp.concatenate(cols, axis=1)


def _topk_indices(queries, points, k, drop_first):
    b, m, _ = queries.shape
    v = points.shape[1]
    vt = jnp.pad(jnp.moveaxis(points, 1, 2), ((0, 0), (0, 5), (0, 0)))
    r = min(m, 256)
    kern = functools.partial(
        _topk_kernel, n_iter=k + (1 if drop_first else 0),
        drop_first=drop_first, v=v)
    return _pallas_call(
        kern,
        grid=(b, m // r),
        in_specs=[
            pl.BlockSpec((1, r, 3), lambda bi, i: (bi, i, 0)),
            pl.BlockSpec((1, 8, v), lambda bi, i: (bi, 0, 0)),
        ],
        out_specs=pl.BlockSpec((1, r, k), lambda bi, i: (bi, i, 0)),
        out_shape=jax.ShapeDtypeStruct((b, m, k), jnp.int32),
    )(queries, vt)


def _sc_gather_async(table, flat_idx):
    n, d = table.shape
    m = flat_idx.shape[0]
    w = 128
    units = 32
    nw = m // (w * units)
    mesh = plsc.VectorSubcoreMesh(core_axis_name="c", subcore_axis_name="s")

    @functools.partial(
        pl.kernel,
        out_type=jax.ShapeDtypeStruct((m, d), table.dtype),
        mesh=mesh,
        scratch_types=[
            pltpu.VMEM((2, w), jnp.int32),
            pltpu.VMEM((2, w, d), table.dtype),
            pltpu.SemaphoreType.DMA((2,)),
            pltpu.SemaphoreType.DMA((2,)),
            pltpu.SemaphoreType.DMA((2,)),
        ])
    def gather_kernel(tab_hbm, i_hbm, o_hbm, ibuf, obuf, isem, gsem, osem):
        ci = jax.lax.axis_index("c")
        si = jax.lax.axis_index("s")
        base = (ci * (units // 2) + si) * nw

        def idx_cp(k, s):
            return pltpu.make_async_copy(
                i_hbm.at[0, pl.ds((base + k) * w, w)], ibuf.at[s], isem.at[s])

        def gat_cp(k, s):
            return pltpu.make_async_copy(
                tab_hbm.at[ibuf.at[s]], obuf.at[s], gsem.at[s])

        def out_cp(k, s):
            return pltpu.make_async_copy(
                obuf.at[s], o_hbm.at[pl.ds((base + k) * w, w)], osem.at[s])

        idx_cp(0, 0).start()
        idx_cp(0, 0).wait()
        gat_cp(0, 0).start()
        if nw > 1:
            idx_cp(1, 1).start()
        for k in range(nw):
            s = k % 2
            if k + 1 < nw:
                idx_cp(k + 1, 1 - s).wait()
                if k >= 1:
                    out_cp(k - 1, 1 - s).wait()
                gat_cp(k + 1, 1 - s).start()
                if k + 2 < nw:
                    idx_cp(k + 2, s).start()
            gat_cp(k, s).wait()
            out_cp(k, s).start()
        out_cp(nw - 1, (nw - 1) % 2).wait()
        if nw > 1:
            out_cp(nw - 2, nw % 2).wait()

    return gather_kernel(table, flat_idx.reshape(1, m))


def _sc_gather(table, flat_idx):
    n, d = table.shape
    m = flat_idx.shape[0]
    w = 128
    if m % (w * 32) == 0:
        return _sc_gather_async(table, flat_idx)
    mesh = plsc.VectorSubcoreMesh(core_axis_name="c", subcore_axis_name="s")

    @functools.partial(
        pl.kernel,
        out_type=jax.ShapeDtypeStruct((m, d), table.dtype),
        mesh=mesh)
    def gather_kernel(tab_hbm, i_hbm, o_hbm):
        def body(i_vmem, o_vmem):
            pltpu.sync_copy(tab_hbm.at[i_vmem.at[0]], o_vmem)

        pltpu.emit_pipeline(
            body,
            grid=(m // w,),
            in_specs=[pl.BlockSpec((1, w), lambda i: (0, i))],
            out_specs=[pl.BlockSpec((w, d), lambda i: (i, 0))],
            core_axis_name=("c", "s"),
            dimension_semantics=(pltpu.PARALLEL,),
        )(i_hbm, o_hbm)

    return gather_kernel(table, flat_idx.reshape(1, m))


def _batched_gather(table, idx):
    b, n, d = table.shape
    off = jnp.arange(b, dtype=jnp.int32).reshape((b,) + (1,) * (idx.ndim - 1))
    flat = (idx + off * n).reshape(-1)
    pw = 256 if d % 256 == 0 else 128
    parts = d // pw
    if parts > 1:
        flat = (flat[:, None] * parts
                + jnp.arange(parts, dtype=jnp.int32)[None, :]).reshape(-1)
    out = _sc_gather(table.reshape(b * n * parts, pw), flat)
    return out.reshape(idx.shape + (d,))


def _pack_bf16(x):
    half = x.shape[-1] // 2
    lo = x[..., :half].astype(jnp.bfloat16)
    hi = x[..., half:].astype(jnp.bfloat16)
    pair = jnp.stack([lo, hi], axis=-1)
    return jax.lax.bitcast_convert_type(pair, jnp.float32)


def _unpack_bf16(p):
    u = jax.lax.bitcast_convert_type(p, jnp.uint32)
    lo = jax.lax.bitcast_convert_type(u << 16, jnp.float32)
    hi = jax.lax.bitcast_convert_type(u & jnp.uint32(0xFFFF0000), jnp.float32)
    return jnp.concatenate([lo, hi], axis=-1)


def _combine_kernel(nbr_ref, ctr_ref, dir_ref, *rest, sup, c, relu, surface):
    if surface:
        (o_ref,) = rest
    else:
        sup_ref, cen_ref, o_ref = rest
    nd = nbr_ref[0] - ctr_ref[0][:, None, :]
    norm = jnp.sqrt(jnp.sum(nd * nd, axis=-1, keepdims=True))
    ndn = nd / jnp.maximum(norm, 1e-12)
    dirs = dir_ref[...]
    dn = jnp.sqrt(jnp.sum(dirs * dirs, axis=0, keepdims=True))
    sd = dirs / jnp.maximum(dn, 1e-12)
    r, n = nd.shape[0], nd.shape[1]
    scw = sd.shape[1]
    a = ndn[..., :8].reshape(r * n, 8).astype(jnp.bfloat16)
    b = jnp.concatenate([sd, jnp.zeros((5, scw), jnp.float32)],
                        axis=0).astype(jnp.bfloat16)
    theta = jnp.dot(a, b, preferred_element_type=jnp.float32)
    theta = jnp.maximum(theta, 0.0).reshape(r, n, scw)
    act = theta if surface else theta * _unpack_bf16(sup_ref[0])
    msum = jnp.max(act, axis=1)
    out = msum[:, 0:c]
    for s in range(1, sup):
        out = out + msum[:, s * c:(s + 1) * c]
    if not surface:
        out = out + cen_ref[0]
    if relu:
        out = jnp.maximum(out, 0.0)
    o_ref[0] = out


def _conv_combine(nbr_xyz, verts_pad, dirs, sup_g, center, relu, r):
    b, v, nn, _ = nbr_xyz.shape
    sc = dirs.shape[1]
    c = sc // _SUP
    surface = sup_g is None
    kern = functools.partial(
        _combine_kernel, sup=_SUP, c=c, relu=relu, surface=surface)
    in_specs = [
        pl.BlockSpec((1, r, nn, 128), lambda bi, i: (bi, i, 0, 0)),
        pl.BlockSpec((1, r, 128), lambda bi, i: (bi, i, 0)),
        pl.BlockSpec((3, sc), lambda bi, i: (0, 0)),
    ]
    args = [nbr_xyz, verts_pad, dirs]
    if not surface:
        in_specs.append(
            pl.BlockSpec((1, r, nn, sc // 2), lambda bi, i: (bi, i, 0, 0)))
        in_specs.append(pl.BlockSpec((1, r, c), lambda bi, i: (bi, i, 0)))
        args += [sup_g, center]
    return _pallas_call(
        kern,
        grid=(b, v // r),
        in_specs=in_specs,
        out_specs=pl.BlockSpec((1, r, c), lambda bi, i: (bi, i, 0)),
        out_shape=jax.ShapeDtypeStruct((b, v, c), jnp.float32),
    )(*args)


def _linear_kernel(x_ref, w_ref, b_ref, o_ref):
    o_ref[0] = (jnp.dot(x_ref[0], w_ref[...],
                        preferred_element_type=jnp.float32) + b_ref[...])


def _linear(x, w, bias, r=256):
    b, m, k = x.shape
    n = w.shape[1]
    r = min(r, m)
    return _pallas_call(
        _linear_kernel,
        grid=(b, m // r),
        in_specs=[
            pl.BlockSpec((1, r, k), lambda bi, i: (bi, i, 0)),
            pl.BlockSpec((k, n), lambda bi, i: (0, 0)),
            pl.BlockSpec((1, n), lambda bi, i: (0, 0)),
        ],
        out_specs=pl.BlockSpec((1, r, n), lambda bi, i: (bi, i, 0)),
        out_shape=jax.ShapeDtypeStruct((b, m, n), jnp.float32),
    )(x, w, bias.reshape(1, n))


def _pool_max_kernel(g_ref, o_ref):
    o_ref[0] = jnp.max(g_ref[0], axis=1).astype(jnp.float32)


def _pool_max(g, r=128):
    b, p, nn, c = g.shape
    r = min(r, p)
    return _pallas_call(
        _pool_max_kernel,
        grid=(b, p // r),
        in_specs=[pl.BlockSpec((1, r, nn, c), lambda bi, i: (bi, i, 0, 0))],
        out_specs=pl.BlockSpec((1, r, c), lambda bi, i: (bi, i, 0)),
        out_shape=jax.ShapeDtypeStruct((b, p, c), jnp.float32),
    )(g)


def _global_max_kernel(x_ref, o_ref):
    o_ref[0, 0] = jnp.max(x_ref[0], axis=0)


def _global_max(x):
    b, v, c = x.shape
    out = _pallas_call(
        _global_max_kernel,
        grid=(b,),
        in_specs=[pl.BlockSpec((1, v, c), lambda bi: (bi, 0, 0))],
        out_specs=pl.BlockSpec((1, 1, c), lambda bi: (bi, 0, 0)),
        out_shape=jax.ShapeDtypeStruct((b, 1, c), jnp.float32),
    )(x)
    return out[:, 0, :]


def _head_kernel(x_ref, w1_ref, b1_ref, w2_ref, b2_ref, w3_ref, b3_ref, o_ref):
    f32 = jnp.float32
    h = jnp.dot(x_ref[0], w1_ref[...], preferred_element_type=f32) + b1_ref[...]
    h = jnp.maximum(h, 0.0)
    h = jnp.dot(h, w2_ref[...], preferred_element_type=f32) + b2_ref[...]
    h = jnp.maximum(h, 0.0)
    o_ref[0] = jnp.dot(h, w3_ref[...], preferred_element_type=f32) + b3_ref[...]


def _head(x, w1, b1, w2, b2, w3, b3, r=256):
    b, m, k = x.shape
    h1 = w1.shape[1]
    n = w3.shape[1]
    return _pallas_call(
        _head_kernel,
        grid=(b, m // r),
        in_specs=[
            pl.BlockSpec((1, r, k), lambda bi, i: (bi, i, 0)),
            pl.BlockSpec((k, h1), lambda bi, i: (0, 0)),
            pl.BlockSpec((1, h1), lambda bi, i: (0, 0)),
            pl.BlockSpec((h1, h1), lambda bi, i: (0, 0)),
            pl.BlockSpec((1, h1), lambda bi, i: (0, 0)),
            pl.BlockSpec((h1, n), lambda bi, i: (0, 0)),
            pl.BlockSpec((1, n), lambda bi, i: (0, 0)),
        ],
        out_specs=pl.BlockSpec((1, r, n), lambda bi, i: (bi, i, 0)),
        out_shape=jax.ShapeDtypeStruct((b, m, n), jnp.float32),
    )(x, w1, b1.reshape(1, h1), w2, b2.reshape(1, h1), w3, b3.reshape(1, n))


def _pad3(x):
    return jnp.pad(x, ((0, 0), (0, 0), (0, 125)))


def _conv_stage(nbr, vpad, fm_in, w, bias, dirs, nidx, out_c, relu, r):
    fo = _linear(fm_in, w, bias)
    cen = fo[:, :, :out_c]
    sup_g = _batched_gather(_pack_bf16(fo[:, :, out_c:]), nidx)
    return _conv_combine(nbr, vpad, dirs, sup_g, cen, relu, r)


def kernel(vertices, onehot, params):
    return _forward_one(vertices, onehot, params)


def _forward_one(vertices, onehot, params):
    b, v, _ = vertices.shape

    nidx1 = _topk_indices(vertices, vertices, _NBR, True)
    vpad = _pad3(vertices)
    nbr1 = _batched_gather(vpad, nidx1)
    fm0 = _conv_combine(nbr1, vpad, params['d0'], None, None, True, 128)
    fm1 = _conv_stage(nbr1, vpad, fm0, params['w1'], params['b1'],
                      params['d1'], nidx1, 128, True, 128)

    sidx1 = jnp.asarray(np.random.RandomState(1).permutation(v)[:v // 4])
    vq1 = vertices[:, sidx1, :]
    pidx1 = nidx1[:, sidx1, :4]
    fmp1 = _pool_max(_batched_gather(fm1, pidx1))

    v2 = v // 4
    nidx2 = _topk_indices(vq1, vq1, _NBR, True)
    vp1pad = _pad3(vq1)
    nbr2 = _batched_gather(vp1pad, nidx2)
    fm2 = _conv_stage(nbr2, vp1pad, fmp1, params['w2'], params['b2'],
                      params['d2'], nidx2, 256, True, 64)
    fm3 = _conv_stage(nbr2, vp1pad, fm2, params['w3'], params['b3'],
                      params['d3'], nidx2, 256, True, 64)

    sidx2 = jnp.asarray(np.random.RandomState(2).permutation(v2)[:v2 // 4])
    vq2 = vq1[:, sidx2, :]
    pidx2 = nidx2[:, sidx2, :4]
    fmp2 = _pool_max(_batched_gather(fm3, pidx2))

    nidx3 = _topk_indices(vq2, vq2, _NBR, True)
    vp2pad = _pad3(vq2)
    nbr3 = _batched_gather(vp2pad, nidx3)
    fm4 = _conv_stage(nbr3, vp2pad, fmp2, params['w4'], params['b4'],
                      params['d4'], nidx3, 512, False, 32)
    fg = _global_max(fm4)

    near1 = _topk_indices(vertices, vq1, 1, False)
    near2 = _topk_indices(vertices, vq2, 1, False)
    f23u = _batched_gather(jnp.concatenate([fm2, fm3], axis=2),
                           near1)[:, :, 0, :]
    f2u = f23u[:, :, :fm2.shape[2]]
    f3u = f23u[:, :, fm2.shape[2]:]
    f4u = _batched_gather(fm4, near2)[:, :, 0, :]

    fuse = jnp.concatenate([
        fm0, fm1, f2u, f3u, f4u,
        jnp.broadcast_to(fg[:, None, :], (b, v, fg.shape[-1])),
        jnp.broadcast_to(onehot[:, None, :], (b, v, onehot.shape[-1])),
    ], axis=2)
    k_fuse = fuse.shape[-1]
    k_pad = -k_fuse % 128
    fuse = jnp.pad(fuse, ((0, 0), (0, 0), (0, k_pad)))
    w1t = jnp.pad(params['cw1'].T, ((0, k_pad), (0, 0)))
    return _head(fuse, w1t, params['cb1'], params['cw2'].T, params['cb2'],
                 params['cw3'].T, params['cb3'])
